# baseline (speedup 1.0000x reference)
import numpy as np

import jax
import jax.numpy as jnp
from jax.experimental import pallas as pl
from jax.experimental.pallas import tpu as pltpu


def _conv1_kernel(x_ref, m1_ref, b1c_ref, mask_ref, o_ref):
    bt = o_ref.shape[0]
    bf16 = jnp.bfloat16
    xq = x_ref[...]
    z1 = jnp.zeros((bt, 1, 56), bf16)
    z3 = jnp.zeros((bt, 3, 56), bf16)
    xqp = jnp.concatenate([z1, xq, z3], axis=1)
    lhs = jnp.concatenate(
        [xqp[:, 0:16, :], xqp[:, 1:17, :], xqp[:, 2:18, :]],
        axis=2)
    a = jnp.dot(lhs.reshape(bt * 16, 168), m1_ref[...],
                preferred_element_type=jnp.float32)
    a = jnp.maximum(jnp.maximum(a[:, 0:256], a[:, 256:512]),
                    jnp.maximum(a[:, 512:768], a[:, 768:1024]))
    a = jnp.maximum(a + b1c_ref[...], 0.0)
    o_ref[...] = (a.reshape(bt, 16, 256) * mask_ref[...]).astype(bf16)


def _conv2_kernel(a_ref, m2_ref, b2c_ref, o_ref):
    bt = o_ref.shape[0]
    bf16 = jnp.bfloat16
    av = a_ref[...]
    zr = jnp.zeros((bt, 1, 512), bf16)
    hp = jnp.concatenate([zr, av, zr], axis=1)
    lhs = jnp.concatenate([hp[:, 0:8, :], hp[:, 1:9, :], hp[:, 2:10, :]],
                          axis=2)
    c = jnp.dot(lhs.reshape(bt * 8, 1536), m2_ref[...],
                preferred_element_type=jnp.float32)
    c = jnp.maximum(jnp.maximum(c[:, 0:256], c[:, 256:512]),
                    jnp.maximum(c[:, 512:768], c[:, 768:1024]))
    c = jnp.maximum(c + b2c_ref[...], 0.0)
    o_ref[...] = c.reshape(bt, 8, 256)


def _fc_logsoftmax_kernel(h_ref, wc_ref, bc_ref, o_ref):
    z = jnp.dot(h_ref[...], wc_ref[...],
                preferred_element_type=jnp.float32) + bc_ref[...]
    m = jnp.max(z, axis=-1, keepdims=True)
    s = z - m
    lse = jnp.log(jnp.sum(jnp.exp(s), axis=-1, keepdims=True))
    o_ref[...] = s - lse


def _m1_paired(taps):
    cols = []
    for hpar in range(2):
        for wpar in range(2):
            m = jnp.zeros((168, 224), jnp.float32)
            for j in range(3):
                for rowpar in range(2):
                    dy = 2 * j + rowpar - 1 - hpar
                    if not 0 <= dy < 3:
                        continue
                    for dx in range(3):
                        s = np.zeros((168, 14), np.float32)
                        for w4 in range(14):
                            win = 2 * w4 + wpar + dx - 1
                            if 0 <= win < 28:
                                s[j * 56 + rowpar * 28 + win, w4] = 1.0
                        m = m + jnp.kron(jnp.asarray(s),
                                         taps[dy * 3 + dx])
            cols.append(jnp.concatenate(
                [m, jnp.zeros((168, 32), jnp.float32)], axis=1))
    return jnp.concatenate(cols, axis=1)


def _m2_paired(taps):
    rows = []
    for j in range(3):
        for h2par in range(2):
            blk = jnp.zeros((256, 1024), jnp.float32)
            cols = []
            for hpar2 in range(2):
                for wpar2 in range(2):
                    m = jnp.zeros((256, 224), jnp.float32)
                    dy = 2 * j + h2par - 1 - hpar2
                    if 0 <= dy < 3:
                        for dx in range(3):
                            s = np.zeros((16, 7), np.float32)
                            for w4 in range(7):
                                win = 2 * w4 + wpar2 + dx - 1
                                if 0 <= win < 16:
                                    s[win, w4] = 1.0
                            m = m + jnp.kron(jnp.asarray(s),
                                             taps[dy * 3 + dx])
                    cols.append(jnp.concatenate(
                        [m, jnp.zeros((256, 32), jnp.float32)], axis=1))
            rows.append(jnp.concatenate(cols, axis=1))
    return jnp.concatenate(rows, axis=0)


def kernel(x, w1, b1, w2, b2, wfc1, bfc1, l0w, l0b, l1w, l1b, l2w, l2b):
    B = x.shape[0]
    bt1 = 128 if B % 128 == 0 else B
    bt2 = 1024 if B % 1024 == 0 else B
    f32 = jnp.float32

    w1taps = [w1[t].reshape(1, 16) for t in range(9)]
    m1 = _m1_paired(w1taps).astype(jnp.bfloat16)
    m2 = _m2_paired([w2[t] for t in range(9)]).astype(jnp.bfloat16)
    b1c = jnp.concatenate([jnp.tile(b1, (1, 14)),
                           jnp.zeros((1, 32), f32)], axis=1)
    b2c = jnp.concatenate([jnp.tile(b2, (1, 7)),
                           jnp.zeros((1, 32), f32)], axis=1)
    mask = np.zeros((1, 16, 256), np.float32)
    mask[:, 0:14, 0:224] = 1.0
    mask = jnp.asarray(mask)

    t1 = l1w @ l2w
    t0 = l0w @ t1
    wc = wfc1 @ t0
    bc = bfc1 @ t0 + l0b @ t1 + l1b @ l2w + l2b
    wc2 = jnp.pad(wc.reshape(7, 7 * 32, 10),
                  ((0, 1), (0, 32), (0, 0))).reshape(2048, 10)

    a1 = pl.pallas_call(
        _conv1_kernel,
        out_shape=jax.ShapeDtypeStruct((B, 16, 256), jnp.bfloat16),
        grid=(B // bt1,),
        in_specs=[
            pl.BlockSpec((bt1, 14, 56), lambda b: (b, 0, 0)),
            pl.BlockSpec((168, 1024), lambda b: (0, 0)),
            pl.BlockSpec((1, 256), lambda b: (0, 0)),
            pl.BlockSpec((1, 16, 256), lambda b: (0, 0, 0)),
        ],
        out_specs=pl.BlockSpec((bt1, 16, 256), lambda b: (b, 0, 0)),
        compiler_params=pltpu.CompilerParams(
            dimension_semantics=("parallel",)),
        cost_estimate=pl.CostEstimate(
            flops=2 * B * 16 * 168 * 1024,
            transcendentals=0,
            bytes_accessed=2 * (B * 784 + B * 16 * 256)),
    )(x.astype(jnp.bfloat16).reshape(B, 14, 56), m1, b1c, mask)

    a2 = pl.pallas_call(
        _conv2_kernel,
        out_shape=jax.ShapeDtypeStruct((B, 8, 256), f32),
        grid=(B // bt1,),
        in_specs=[
            pl.BlockSpec((bt1, 8, 512), lambda b: (b, 0, 0)),
            pl.BlockSpec((1536, 1024), lambda b: (0, 0)),
            pl.BlockSpec((1, 256), lambda b: (0, 0)),
        ],
        out_specs=pl.BlockSpec((bt1, 8, 256), lambda b: (b, 0, 0)),
        compiler_params=pltpu.CompilerParams(
            dimension_semantics=("parallel",)),
        cost_estimate=pl.CostEstimate(
            flops=2 * B * 8 * 1536 * 1024,
            transcendentals=0,
            bytes_accessed=2 * B * 8 * 512 + 4 * B * 8 * 256),
    )(a1.reshape(B, 8, 512), m2, b2c)

    h = a2.reshape(B, 2048)
    out = pl.pallas_call(
        _fc_logsoftmax_kernel,
        out_shape=jax.ShapeDtypeStruct((B, 10), f32),
        grid=(B // bt2,),
        in_specs=[
            pl.BlockSpec((bt2, 2048), lambda b: (b, 0)),
            pl.BlockSpec((2048, 10), lambda b: (0, 0)),
            pl.BlockSpec((1, 10), lambda b: (0, 0)),
        ],
        out_specs=pl.BlockSpec((bt2, 10), lambda b: (b, 0)),
        compiler_params=pltpu.CompilerParams(
            dimension_semantics=("parallel",)),
        cost_estimate=pl.CostEstimate(
            flops=2 * B * 2048 * 10,
            transcendentals=B * 10,
            bytes_accessed=4 * (B * 2048 + B * 10)),
    )(h, wc2, bc)
    return out


# speedup vs baseline: 1.0492x; 1.0492x over previous
import numpy as np

import jax
import jax.numpy as jnp
from jax.experimental import pallas as pl
from jax.experimental.pallas import tpu as pltpu


def _conv_stack_kernel(x_ref, m1_ref, b1q_ref, mask_ref, m2_ref, b2c_ref,
                       o_ref):
    bt = o_ref.shape[0]
    bf16 = jnp.bfloat16
    f32 = jnp.float32
    xq = x_ref[...]
    z1 = jnp.zeros((bt, 1, 112), bf16)
    z2 = jnp.zeros((bt, 2, 112), bf16)
    xqp = jnp.concatenate([z1, xq, z2], axis=1)
    lhs = jnp.concatenate(
        [xqp[:, 0:8, :], xqp[:, 1:9, :], xqp[:, 2:10, :]],
        axis=2)
    y = jnp.dot(lhs.reshape(bt * 8, 336), m1_ref[...],
                preferred_element_type=f32)
    p0 = jnp.maximum(jnp.maximum(y[:, 0:256], y[:, 256:512]),
                     jnp.maximum(y[:, 512:768], y[:, 768:1024]))
    p1 = jnp.maximum(jnp.maximum(y[:, 1024:1280], y[:, 1280:1536]),
                     jnp.maximum(y[:, 1536:1792], y[:, 1792:2048]))
    a = jnp.concatenate([p0, p1], axis=1)
    a = jnp.maximum(a + b1q_ref[...], 0.0)
    a = (a.reshape(bt, 8, 512) * mask_ref[...]).astype(bf16)
    zr = jnp.zeros((bt, 1, 512), bf16)
    hp = jnp.concatenate([zr, a, zr], axis=1)
    lhs2 = jnp.concatenate([hp[:, 0:8, :], hp[:, 1:9, :], hp[:, 2:10, :]],
                           axis=2)
    c = jnp.dot(lhs2.reshape(bt * 8, 1536), m2_ref[...],
                preferred_element_type=f32)
    c = jnp.maximum(jnp.maximum(c[:, 0:256], c[:, 256:512]),
                    jnp.maximum(c[:, 512:768], c[:, 768:1024]))
    c = jnp.maximum(c + b2c_ref[...], 0.0)
    o_ref[...] = c.reshape(bt, 8, 256)


def _fc_logsoftmax_kernel(h_ref, wc_ref, bc_ref, o_ref):
    z = jnp.dot(h_ref[...], wc_ref[...],
                preferred_element_type=jnp.float32) + bc_ref[...]
    m = jnp.max(z, axis=-1, keepdims=True)
    s = z - m
    lse = jnp.log(jnp.sum(jnp.exp(s), axis=-1, keepdims=True))
    o_ref[...] = s - lse


def _m1_quad(taps):
    cols = []
    for h2par in range(2):
        for hpar in range(2):
            for wpar in range(2):
                m = jnp.zeros((336, 224), jnp.float32)
                for j in range(3):
                    for quadpos in range(4):
                        dy = 4 * j + quadpos - 2 * h2par - hpar - 3
                        if not 0 <= dy < 3:
                            continue
                        for dx in range(3):
                            s = np.zeros((336, 14), np.float32)
                            for w4 in range(14):
                                win = 2 * w4 + wpar + dx - 1
                                if 0 <= win < 28:
                                    s[j * 112 + quadpos * 28 + win,
                                      w4] = 1.0
                            m = m + jnp.kron(jnp.asarray(s),
                                             taps[dy * 3 + dx])
                cols.append(jnp.concatenate(
                    [m, jnp.zeros((336, 32), jnp.float32)], axis=1))
    return jnp.concatenate(cols, axis=1)


def _m2_paired(taps):
    rows = []
    for j in range(3):
        for h2par in range(2):
            cols = []
            for hpar2 in range(2):
                for wpar2 in range(2):
                    m = jnp.zeros((256, 224), jnp.float32)
                    dy = 2 * j + h2par - 1 - hpar2
                    if 0 <= dy < 3:
                        for dx in range(3):
                            s = np.zeros((16, 7), np.float32)
                            for w4 in range(7):
                                win = 2 * w4 + wpar2 + dx - 1
                                if 0 <= win < 16:
                                    s[win, w4] = 1.0
                            m = m + jnp.kron(jnp.asarray(s),
                                             taps[dy * 3 + dx])
                    cols.append(jnp.concatenate(
                        [m, jnp.zeros((256, 32), jnp.float32)], axis=1))
            rows.append(jnp.concatenate(cols, axis=1))
    return jnp.concatenate(rows, axis=0)


def kernel(x, w1, b1, w2, b2, wfc1, bfc1, l0w, l0b, l1w, l1b, l2w, l2b):
    B = x.shape[0]
    bt1 = 128 if B % 128 == 0 else B
    bt2 = 1024 if B % 1024 == 0 else B
    f32 = jnp.float32

    w1taps = [w1[t].reshape(1, 16) for t in range(9)]
    m1 = _m1_quad(w1taps).astype(jnp.bfloat16)
    m2 = _m2_paired([w2[t] for t in range(9)]).astype(jnp.bfloat16)
    b1h = jnp.concatenate([jnp.tile(b1, (1, 14)),
                           jnp.zeros((1, 32), f32)], axis=1)
    b1q = jnp.tile(b1h, (1, 2))
    b2c = jnp.concatenate([jnp.tile(b2, (1, 7)),
                           jnp.zeros((1, 32), f32)], axis=1)
    mask = np.zeros((1, 8, 512), np.float32)
    mask[:, 0:7, 0:224] = 1.0
    mask[:, 0:7, 256:480] = 1.0
    mask = jnp.asarray(mask)

    t1 = l1w @ l2w
    t0 = l0w @ t1
    wc = wfc1 @ t0
    bc = bfc1 @ t0 + l0b @ t1 + l1b @ l2w + l2b
    wc2 = jnp.pad(wc.reshape(7, 7 * 32, 10),
                  ((0, 1), (0, 32), (0, 0))).reshape(2048, 10)

    conv_out = pl.pallas_call(
        _conv_stack_kernel,
        out_shape=jax.ShapeDtypeStruct((B, 8, 256), f32),
        grid=(B // bt1,),
        in_specs=[
            pl.BlockSpec((bt1, 7, 112), lambda b: (b, 0, 0)),
            pl.BlockSpec((336, 2048), lambda b: (0, 0)),
            pl.BlockSpec((1, 512), lambda b: (0, 0)),
            pl.BlockSpec((1, 8, 512), lambda b: (0, 0, 0)),
            pl.BlockSpec((1536, 1024), lambda b: (0, 0)),
            pl.BlockSpec((1, 256), lambda b: (0, 0)),
        ],
        out_specs=pl.BlockSpec((bt1, 8, 256), lambda b: (b, 0, 0)),
        compiler_params=pltpu.CompilerParams(
            dimension_semantics=("parallel",)),
        cost_estimate=pl.CostEstimate(
            flops=2 * B * 8 * (336 * 2048 + 1536 * 1024),
            transcendentals=0,
            bytes_accessed=2 * B * 784 + 4 * B * 8 * 256),
    )(x.astype(jnp.bfloat16).reshape(B, 7, 112), m1, b1q, mask, m2, b2c)

    h = conv_out.reshape(B, 2048)
    out = pl.pallas_call(
        _fc_logsoftmax_kernel,
        out_shape=jax.ShapeDtypeStruct((B, 10), f32),
        grid=(B // bt2,),
        in_specs=[
            pl.BlockSpec((bt2, 2048), lambda b: (b, 0)),
            pl.BlockSpec((2048, 10), lambda b: (0, 0)),
            pl.BlockSpec((1, 10), lambda b: (0, 0)),
        ],
        out_specs=pl.BlockSpec((bt2, 10), lambda b: (b, 0)),
        compiler_params=pltpu.CompilerParams(
            dimension_semantics=("parallel",)),
        cost_estimate=pl.CostEstimate(
            flops=2 * B * 2048 * 10,
            transcendentals=B * 10,
            bytes_accessed=4 * (B * 2048 + B * 10)),
    )(h, wc2, bc)
    return out


# speedup vs baseline: 1.1296x; 1.0767x over previous
import numpy as np

import jax
import jax.numpy as jnp
from jax.experimental import pallas as pl
from jax.experimental.pallas import tpu as pltpu


def _conv_stack_kernel(x_ref, m1_ref, b1t_ref, m2_ref, b2t_ref, mask_ref,
                       o_ref):
    bt = o_ref.shape[0]
    f32 = jnp.float32
    bf16 = jnp.bfloat16
    xq = x_ref[...]
    z1 = jnp.zeros((bt, 1, 56), bf16)
    z3 = jnp.zeros((bt, 3, 56), bf16)
    xqp = jnp.concatenate([z1, xq, z3], axis=1)
    lhs = jnp.concatenate(
        [xqp[:, 0:16, :], xqp[:, 1:17, :], xqp[:, 2:18, :]],
        axis=2)
    a = jnp.dot(lhs.reshape(bt * 16, 168), m1_ref[...],
                preferred_element_type=f32)
    a = jnp.maximum(a + b1t_ref[...], 0.0)
    a = jnp.maximum(jnp.maximum(a[:, 0:256], a[:, 256:512]),
                    jnp.maximum(a[:, 512:768], a[:, 768:1024]))
    a = (a.reshape(bt, 16, 256) * mask_ref[...]).astype(bf16)
    zr = jnp.zeros((bt, 1, 256), bf16)
    hp = jnp.concatenate([zr, a, zr], axis=1)
    lhs2 = jnp.concatenate([hp[:, 0:16, :], hp[:, 1:17, :], hp[:, 2:18, :]],
                           axis=2)
    c = jnp.dot(lhs2.reshape(bt * 16, 768), m2_ref[...],
                preferred_element_type=f32)
    c = jnp.maximum(c + b2t_ref[...], 0.0)
    c = jnp.maximum(c[:, 0:256], c[:, 256:512])
    c = jnp.max(c.reshape(bt * 8, 2, 256), axis=1)
    o_ref[...] = c.reshape(bt, 8, 256)


def _fc_logsoftmax_kernel(h_ref, wc_ref, bc_ref, o_ref):
    z = jnp.dot(h_ref[...], wc_ref[...],
                preferred_element_type=jnp.float32) + bc_ref[...]
    m = jnp.max(z, axis=-1, keepdims=True)
    s = z - m
    lse = jnp.log(jnp.sum(jnp.exp(s), axis=-1, keepdims=True))
    o_ref[...] = s - lse


def _m1_paired(taps):
    cols = []
    for hpar in range(2):
        for wpar in range(2):
            m = jnp.zeros((168, 224), jnp.float32)
            for j in range(3):
                for rowpar in range(2):
                    dy = 2 * j + rowpar - 1 - hpar
                    if not 0 <= dy < 3:
                        continue
                    for dx in range(3):
                        s = np.zeros((168, 14), np.float32)
                        for w4 in range(14):
                            win = 2 * w4 + wpar + dx - 1
                            if 0 <= win < 28:
                                s[j * 56 + rowpar * 28 + win, w4] = 1.0
                        m = m + jnp.kron(jnp.asarray(s),
                                         taps[dy * 3 + dx])
            cols.append(jnp.concatenate(
                [m, jnp.zeros((168, 32), jnp.float32)], axis=1))
    return jnp.concatenate(cols, axis=1)


def _banded(taps, n_slots, n_w, cin, cout):
    dy_blocks = []
    for dy in range(3):
        par_blocks = []
        for par in range(2):
            m = jnp.zeros((n_slots * cin, n_w * cout), jnp.float32)
            for dx in range(3):
                s = np.zeros((n_slots, n_w), np.float32)
                for w4 in range(n_w):
                    win = 2 * w4 + par + dx - 1
                    if 0 <= win < n_slots:
                        s[win, w4] = 1.0
                m = m + jnp.kron(jnp.asarray(s), taps[dy * 3 + dx])
            pad = jnp.zeros((n_slots * cin, 256 - n_w * cout), jnp.float32)
            par_blocks.append(jnp.concatenate([m, pad], axis=1))
        dy_blocks.append(jnp.concatenate(par_blocks, axis=1))
    return jnp.concatenate(dy_blocks, axis=0)


def kernel(x, w1, b1, w2, b2, wfc1, bfc1, l0w, l0b, l1w, l1b, l2w, l2b):
    B = x.shape[0]
    bt1 = 128 if B % 128 == 0 else B
    bt2 = 1024 if B % 1024 == 0 else B

    w1taps = [w1[t].reshape(1, 16) for t in range(9)]
    m1 = _m1_paired(w1taps).astype(jnp.bfloat16)
    m2 = _banded([w2[t] for t in range(9)],
                 16, 7, 16, 32).astype(jnp.bfloat16)
    b1t = jnp.tile(b1, (1, 64))
    b2t = jnp.tile(b2, (1, 16))
    mask = np.zeros((1, 16, 256), np.float32)
    mask[:, 0:14, 0:224] = 1.0
    mask = jnp.asarray(mask)

    t1 = l1w @ l2w
    t0 = l0w @ t1
    wc = wfc1 @ t0
    bc = bfc1 @ t0 + l0b @ t1 + l1b @ l2w + l2b
    wc2 = jnp.pad(wc.reshape(7, 7 * 32, 10),
                  ((0, 1), (0, 32), (0, 0))).reshape(2048, 10)

    conv_out = pl.pallas_call(
        _conv_stack_kernel,
        out_shape=jax.ShapeDtypeStruct((B, 8, 256), jnp.float32),
        grid=(B // bt1,),
        in_specs=[
            pl.BlockSpec((bt1, 14, 56), lambda b: (b, 0, 0)),
            pl.BlockSpec((168, 1024), lambda b: (0, 0)),
            pl.BlockSpec((1, 1024), lambda b: (0, 0)),
            pl.BlockSpec((768, 512), lambda b: (0, 0)),
            pl.BlockSpec((1, 512), lambda b: (0, 0)),
            pl.BlockSpec((1, 16, 256), lambda b: (0, 0, 0)),
        ],
        out_specs=pl.BlockSpec((bt1, 8, 256), lambda b: (b, 0, 0)),
        compiler_params=pltpu.CompilerParams(
            dimension_semantics=("parallel",)),
        cost_estimate=pl.CostEstimate(
            flops=2 * B * (16 * 168 * 1024 + 16 * 768 * 512),
            transcendentals=0,
            bytes_accessed=2 * B * 784 + 4 * B * 8 * 256),
    )(x.astype(jnp.bfloat16).reshape(B, 14, 56), m1, b1t, m2, b2t, mask)

    h = conv_out.reshape(B, 2048)
    out = pl.pallas_call(
        _fc_logsoftmax_kernel,
        out_shape=jax.ShapeDtypeStruct((B, 10), jnp.float32),
        grid=(B // bt2,),
        in_specs=[
            pl.BlockSpec((bt2, 2048), lambda b: (b, 0)),
            pl.BlockSpec((2048, 10), lambda b: (0, 0)),
            pl.BlockSpec((1, 10), lambda b: (0, 0)),
        ],
        out_specs=pl.BlockSpec((bt2, 10), lambda b: (b, 0)),
        compiler_params=pltpu.CompilerParams(
            dimension_semantics=("parallel",)),
        cost_estimate=pl.CostEstimate(
            flops=2 * B * 2048 * 10,
            transcendentals=B * 10,
            bytes_accessed=4 * (B * 2048 + B * 10)),
    )(h, wc2, bc)
    return out
